# Initial kernel scaffold; baseline (speedup 1.0000x reference)
import jax
import jax.numpy as jnp
from jax.experimental import pallas as pl


def kernel(entity_embedding, relation_embedding, sample):
    raise NotImplementedError("write your pallas kernel here")


# speedup vs baseline: 2.0214x; 2.0214x over previous
import functools

import jax
import jax.numpy as jnp
from jax import lax
from jax.experimental import pallas as pl
from jax.experimental.pallas import tpu as pltpu
from jax.experimental.pallas import tpu_sc as plsc

GAMMA = 12.0
HIDDEN = 128
BATCH = 16384
NUM_WORKERS = 32
SAMPLES_PER_W = BATCH // NUM_WORKERS
CHUNK = 128
NCHUNK = SAMPLES_PER_W // CHUNK
GRP = 16

_mesh = plsc.VectorSubcoreMesh(core_axis_name="c", subcore_axis_name="s")


@functools.partial(
    pl.kernel,
    mesh=_mesh,
    out_type=jax.ShapeDtypeStruct((BATCH,), jnp.float32),
    compiler_params=pltpu.CompilerParams(needs_layout_passes=False),
    scratch_types=[
        pltpu.VMEM((NCHUNK, CHUNK), jnp.int32),
        pltpu.VMEM((NCHUNK, CHUNK), jnp.int32),
        pltpu.VMEM((NCHUNK, CHUNK), jnp.int32),
        pltpu.VMEM((CHUNK, HIDDEN), jnp.float32),
        pltpu.VMEM((CHUNK, HIDDEN), jnp.float32),
        pltpu.VMEM((CHUNK, HIDDEN), jnp.float32),
        pltpu.VMEM((GRP * 16,), jnp.float32),
        pltpu.VMEM((SAMPLES_PER_W,), jnp.float32),
        pltpu.SemaphoreType.DMA,
    ],
)
def _score_kernel(ent_hbm, rel_hbm, hidx_hbm, ridx_hbm, tidx_hbm, out_hbm,
                  ih, ir, it, hv, rv, tv, accbuf, outv, sem):
    wid = lax.axis_index("s") * 2 + lax.axis_index("c")
    base = wid * SAMPLES_PER_W

    for c in range(NCHUNK):
        off = base + c * CHUNK
        pltpu.sync_copy(hidx_hbm.at[pl.ds(off, CHUNK)], ih.at[c])
        pltpu.sync_copy(ridx_hbm.at[pl.ds(off, CHUNK)], ir.at[c])
        pltpu.sync_copy(tidx_hbm.at[pl.ds(off, CHUNK)], it.at[c])

    for c in range(NCHUNK):
        cp_h = pltpu.async_copy(ent_hbm.at[ih.at[c]], hv, sem)
        cp_r = pltpu.async_copy(rel_hbm.at[ir.at[c]], rv, sem)
        cp_t = pltpu.async_copy(ent_hbm.at[it.at[c]], tv, sem)
        cp_h.wait()
        cp_r.wait()
        cp_t.wait()

        lane = lax.iota(jnp.int32, 16)

        def g_body(g, _):
            def s_body(i, v):
                s = g * GRP + i
                acc = jnp.zeros((16,), jnp.float32)
                for j in range(HIDDEN // 16):
                    dh = hv[s, pl.ds(j * 16, 16)]
                    dr = rv[s, pl.ds(j * 16, 16)]
                    dt = tv[s, pl.ds(j * 16, 16)]
                    acc = acc + jnp.abs(dh + dr - dt)
                return jnp.where(lane == i, jnp.sum(acc), v)

            v = lax.fori_loop(0, GRP, s_body, jnp.zeros((16,), jnp.float32))
            outv[pl.ds(c * CHUNK + g * GRP, GRP)] = GAMMA - v
            return 0

        lax.fori_loop(0, CHUNK // GRP, g_body, 0)

    pltpu.sync_copy(outv, out_hbm.at[pl.ds(base, SAMPLES_PER_W)])


def kernel(entity_embedding, relation_embedding, sample):
    h = sample[:, 0].astype(jnp.int32)
    r = sample[:, 1].astype(jnp.int32)
    t = sample[:, 2].astype(jnp.int32)
    out = _score_kernel(entity_embedding, relation_embedding, h, r, t)
    return out.reshape(BATCH, 1)
